# Initial kernel scaffold; baseline (speedup 1.0000x reference)
import jax
import jax.numpy as jnp
from jax.experimental import pallas as pl


def kernel(x, embed):
    raise NotImplementedError("write your pallas kernel here")


# speedup vs baseline: 1.0735x; 1.0735x over previous
import functools

import jax
import jax.numpy as jnp
from jax import lax
from jax.experimental import pallas as pl
from jax.experimental.pallas import tpu as pltpu
from jax.experimental.pallas import tpu_sc as plsc

DIM = 64
N_CODES = 1024
BATCH = 16
HW = 1024
N_TOKENS = BATCH * HW
INV_TOKENS = 1.0 / N_TOKENS
INV_HWD = 1.0 / (HW * DIM)

NW = 32
BPW = N_TOKENS // NW


def _vq_tc_body(x_ref, embed_ref, ind_ref, diff_ref, perp_ref, counts_ref):
    b = pl.program_id(0)
    x_b = x_ref[0]
    embed = embed_ref[...]
    z2 = jnp.sum(x_b * x_b, axis=0)
    e2 = jnp.sum(embed * embed, axis=0)
    prod = lax.dot_general(x_b, embed, (((0,), (0,)), ((), ())),
                           preferred_element_type=jnp.float32)
    dist = (z2[:, None] - 2.0 * prod) + e2[None, :]
    neg = -dist
    m = jnp.max(neg, axis=1, keepdims=True)
    col_ids = lax.broadcasted_iota(jnp.int32, (HW, N_CODES), 1)
    ind = jnp.min(jnp.where(neg == m, col_ids, N_CODES), axis=1)
    ind_ref[0, 0, :] = ind
    diff_ref[0] = jnp.sum(-m, axis=0, keepdims=True) * INV_HWD
    onehot = (ind[:, None] == col_ids).astype(jnp.float32)
    counts = jnp.sum(onehot, axis=0)

    @pl.when(b == 0)
    def _init():
        counts_ref[...] = counts

    @pl.when(b > 0)
    def _acc():
        counts_ref[...] = counts_ref[...] + counts

    @pl.when(b == BATCH - 1)
    def _fin():
        avg = (counts_ref[...] * INV_TOKENS).reshape(1, N_CODES)
        ent = jnp.sum(avg * jnp.log(avg + 1e-10), axis=1, keepdims=True)
        perp_ref[...] = jnp.exp(-ent)


def _vq_tc(x3, embed):
    return pl.pallas_call(
        _vq_tc_body,
        grid=(BATCH,),
        in_specs=[
            pl.BlockSpec((1, DIM, HW), lambda b: (b, 0, 0)),
            pl.BlockSpec((DIM, N_CODES), lambda b: (0, 0)),
        ],
        out_specs=[
            pl.BlockSpec((1, 1, HW), lambda b: (b, 0, 0)),
            pl.BlockSpec((1, 1, 1), lambda b: (b, 0, 0)),
            pl.BlockSpec((1, 1), lambda b: (0, 0)),
        ],
        out_shape=[
            jax.ShapeDtypeStruct((BATCH, 1, HW), jnp.int32),
            jax.ShapeDtypeStruct((BATCH, 1, 1), jnp.float32),
            jax.ShapeDtypeStruct((1, 1), jnp.float32),
        ],
        scratch_shapes=[pltpu.VMEM((N_CODES,), jnp.float32)],
    )(x3, embed)


ROW = 128


def _sc_gather(idx_flat, table):
    mesh = plsc.VectorSubcoreMesh(core_axis_name="c", subcore_axis_name="s")

    @functools.partial(
        pl.kernel, mesh=mesh,
        out_type=jax.ShapeDtypeStruct((N_TOKENS, ROW), jnp.float32),
        scratch_types=[
            pltpu.VMEM((BPW,), jnp.int32),
            pltpu.VMEM((BPW, ROW), jnp.float32),
            pltpu.SemaphoreType.DMA,
        ],
    )
    def k(idx_hbm, table_hbm, out_hbm, idx_v, rows_v, sem):
        wid = lax.axis_index("s") * 2 + lax.axis_index("c")
        base = wid * BPW
        pltpu.sync_copy(idx_hbm.at[pl.ds(base, BPW)], idx_v)
        pltpu.async_copy(table_hbm.at[idx_v], rows_v, sem).wait()
        pltpu.sync_copy(rows_v, out_hbm.at[pl.ds(base, BPW)])

    return k(idx_flat, table)


def kernel(x, embed):
    x3 = x.reshape(BATCH, DIM, HW)
    ind3, diff3, perp2 = _vq_tc(x3, embed)
    table = jnp.pad(embed.T, ((0, 0), (0, ROW - DIM)))
    qwide = _sc_gather(ind3.reshape(N_TOKENS), table)
    quantize = qwide[:, :DIM].reshape(BATCH, 32, 32, DIM).transpose(0, 3, 1, 2)
    return quantize, diff3.reshape(BATCH, 1), perp2.reshape(())


# speedup vs baseline: 1.3218x; 1.2312x over previous
import functools

import jax
import jax.numpy as jnp
from jax import lax
from jax.experimental import pallas as pl
from jax.experimental.pallas import tpu as pltpu
from jax.experimental.pallas import tpu_sc as plsc

DIM = 64
N_CODES = 1024
BATCH = 16
HW = 1024
N_TOKENS = BATCH * HW
INV_TOKENS = 1.0 / N_TOKENS
INV_HWD = 1.0 / (HW * DIM)

HALF = BATCH // 2
NW = 32
BPW = (HALF * HW) // NW


def _vq_tc_body(x_ref, embed_ref, ind_ref, diff_ref, counts_out_ref,
                counts_ref):
    b = pl.program_id(0)
    x_b = x_ref[0]
    embed = embed_ref[...]
    z2 = jnp.sum(x_b * x_b, axis=0)
    e2 = jnp.sum(embed * embed, axis=0)
    prod = lax.dot_general(embed, x_b, (((0,), (0,)), ((), ())),
                           preferred_element_type=jnp.float32)
    neg = (2.0 * prod - z2[None, :]) - e2[:, None]
    m = jnp.max(neg, axis=0)
    ind = jnp.argmax(neg, axis=0).astype(jnp.int32)
    ind_ref[0, 0, :] = ind
    diff_ref[0] = jnp.sum(-m.reshape(1, HW), axis=1, keepdims=True) * INV_HWD
    onehot = (neg == m[None, :]).astype(jnp.float32)
    counts = lax.dot_general(onehot, jnp.ones((HW, 1), jnp.float32),
                             (((1,), (0,)), ((), ())),
                             preferred_element_type=jnp.float32)

    @pl.when(b == 0)
    def _init():
        counts_ref[...] = counts

    @pl.when(b > 0)
    def _acc():
        counts_ref[...] = counts_ref[...] + counts

    @pl.when(b == HALF - 1)
    def _fin():
        counts_out_ref[...] = counts_ref[...]


def _vq_tc(x3_half, embed):
    return pl.pallas_call(
        _vq_tc_body,
        grid=(HALF,),
        in_specs=[
            pl.BlockSpec((1, DIM, HW), lambda b: (b, 0, 0)),
            pl.BlockSpec((DIM, N_CODES), lambda b: (0, 0)),
        ],
        out_specs=[
            pl.BlockSpec((1, 1, HW), lambda b: (b, 0, 0)),
            pl.BlockSpec((1, 1, 1), lambda b: (b, 0, 0)),
            pl.BlockSpec((N_CODES, 1), lambda b: (0, 0)),
        ],
        out_shape=[
            jax.ShapeDtypeStruct((HALF, 1, HW), jnp.int32),
            jax.ShapeDtypeStruct((HALF, 1, 1), jnp.float32),
            jax.ShapeDtypeStruct((N_CODES, 1), jnp.float32),
        ],
        scratch_shapes=[pltpu.VMEM((N_CODES, 1), jnp.float32)],
    )(x3_half, embed)


def _perp_tc_body(ca_ref, cb_ref, perp_ref):
    avg = (ca_ref[...] + cb_ref[...]) * INV_TOKENS
    ent = jnp.sum(avg * jnp.log(avg + 1e-10), axis=0, keepdims=True)
    perp_ref[...] = jnp.exp(-ent)


def _perp_tc(counts_a, counts_b):
    return pl.pallas_call(
        _perp_tc_body,
        out_shape=jax.ShapeDtypeStruct((1, 1), jnp.float32),
    )(counts_a, counts_b)


def _sc_gather(idx_flat, embed):
    mesh = plsc.VectorSubcoreMesh(core_axis_name="c", subcore_axis_name="s")
    per_batch = HW // BPW

    @functools.partial(
        pl.kernel, mesh=mesh,
        out_type=jax.ShapeDtypeStruct((HALF, DIM, HW), jnp.float32),
        scratch_types=[
            pltpu.VMEM((DIM, N_CODES), jnp.float32),
            pltpu.VMEM((BPW,), jnp.int32),
            pltpu.VMEM((DIM, BPW), jnp.float32),
            pltpu.SemaphoreType.DMA,
        ],
        compiler_params=pltpu.CompilerParams(
            use_tc_tiling_on_sc=False, needs_layout_passes=False),
    )
    def k(idx_hbm, embed_hbm, out_hbm, embed_v, idx_v, vals_v, sem):
        wid = lax.axis_index("s") * 2 + lax.axis_index("c")
        b = wid // per_batch
        hw0 = (wid % per_batch) * BPW
        copy = pltpu.async_copy(embed_hbm, embed_v, sem)
        pltpu.sync_copy(idx_hbm.at[pl.ds(b * HW + hw0, BPW)], idx_v)
        copy.wait()

        def per_group(g, _):
            idx16 = idx_v[pl.ds(g * 16, 16)]

            def per_dim(d, _):
                dvec = jnp.full((16,), 0, jnp.int32) + d
                vals_v[d, pl.ds(g * 16, 16)] = plsc.load_gather(
                    embed_v, [dvec, idx16])
                return 0

            lax.fori_loop(0, DIM, per_dim, 0, unroll=4)
            return 0

        lax.fori_loop(0, BPW // 16, per_group, 0)
        pltpu.sync_copy(vals_v, out_hbm.at[b, :, pl.ds(hw0, BPW)])

    return k(idx_flat, embed)


def kernel(x, embed):
    x3 = x.reshape(BATCH, DIM, HW)
    ind_a, diff_a, counts_a = _vq_tc(x3[:HALF], embed)
    q_a = _sc_gather(ind_a.reshape(HALF * HW), embed)
    ind_b, diff_b, counts_b = _vq_tc(x3[HALF:], embed)
    q_b = _sc_gather(ind_b.reshape(HALF * HW), embed)
    perp = _perp_tc(counts_a, counts_b)
    quantize = jnp.concatenate([q_a, q_b], axis=0).reshape(BATCH, DIM, 32, 32)
    diff = jnp.concatenate([diff_a, diff_b], axis=0).reshape(BATCH, 1)
    return quantize, diff, perp.reshape(())


# speedup vs baseline: 1.4237x; 1.0771x over previous
import functools

import jax
import jax.numpy as jnp
from jax import lax
from jax.experimental import pallas as pl
from jax.experimental.pallas import tpu as pltpu
from jax.experimental.pallas import tpu_sc as plsc

DIM = 64
N_CODES = 1024
BATCH = 16
HW = 1024
N_TOKENS = BATCH * HW
INV_TOKENS = 1.0 / N_TOKENS
INV_HWD = 1.0 / (HW * DIM)

NW = 32
BPW = N_TOKENS // NW


def _vq_tc_body(x_ref, embed_ref, ind_ref, diff_ref, perp_ref, counts_ref):
    b = pl.program_id(0)
    x_b = x_ref[0]
    embed = embed_ref[...]
    z2 = jnp.sum(x_b * x_b, axis=0)
    e2 = jnp.sum(embed * embed, axis=0)
    prod = lax.dot_general(embed, x_b, (((0,), (0,)), ((), ())),
                           preferred_element_type=jnp.float32)
    neg = (2.0 * prod - z2[None, :]) - e2[:, None]
    m = jnp.max(neg, axis=0)
    ind = jnp.argmax(neg, axis=0).astype(jnp.int32)
    ind_ref[0, 0, :] = ind
    diff_ref[0] = jnp.sum(-m.reshape(1, HW), axis=1, keepdims=True) * INV_HWD
    onehot = (neg == m[None, :]).astype(jnp.float32)
    counts = lax.dot_general(onehot, jnp.ones((HW, 1), jnp.float32),
                             (((1,), (0,)), ((), ())),
                             preferred_element_type=jnp.float32)

    @pl.when(b == 0)
    def _init():
        counts_ref[...] = counts

    @pl.when(b > 0)
    def _acc():
        counts_ref[...] = counts_ref[...] + counts

    @pl.when(b == BATCH - 1)
    def _fin():
        avg = counts_ref[...] * INV_TOKENS
        ent = jnp.sum(avg * jnp.log(avg + 1e-10), axis=0, keepdims=True)
        perp_ref[...] = jnp.exp(-ent)


def _vq_tc(x3, embed):
    return pl.pallas_call(
        _vq_tc_body,
        grid=(BATCH,),
        in_specs=[
            pl.BlockSpec((1, DIM, HW), lambda b: (b, 0, 0)),
            pl.BlockSpec((DIM, N_CODES), lambda b: (0, 0)),
        ],
        out_specs=[
            pl.BlockSpec((1, 1, HW), lambda b: (b, 0, 0)),
            pl.BlockSpec((1, 1, 1), lambda b: (b, 0, 0)),
            pl.BlockSpec((1, 1), lambda b: (0, 0)),
        ],
        out_shape=[
            jax.ShapeDtypeStruct((BATCH, 1, HW), jnp.int32),
            jax.ShapeDtypeStruct((BATCH, 1, 1), jnp.float32),
            jax.ShapeDtypeStruct((1, 1), jnp.float32),
        ],
        scratch_shapes=[pltpu.VMEM((N_CODES, 1), jnp.float32)],
        compiler_params=pltpu.CompilerParams(allow_input_fusion=[0]),
    )(x3, embed)


def _sc_gather(idx_flat, embed):
    mesh = plsc.VectorSubcoreMesh(core_axis_name="c", subcore_axis_name="s")

    @functools.partial(
        pl.kernel, mesh=mesh,
        out_type=jax.ShapeDtypeStruct((BATCH, DIM, HW), jnp.float32),
        scratch_types=[
            pltpu.VMEM((DIM, N_CODES), jnp.float32),
            pltpu.VMEM((BPW,), jnp.int32),
            pltpu.VMEM((DIM, BPW), jnp.float32),
            pltpu.SemaphoreType.DMA,
        ],
        compiler_params=pltpu.CompilerParams(
            use_tc_tiling_on_sc=False, needs_layout_passes=False),
    )
    def k(idx_hbm, embed_hbm, out_hbm, embed_v, idx_v, vals_v, sem):
        wid = lax.axis_index("s") * 2 + lax.axis_index("c")
        b = wid // 2
        hw0 = (wid % 2) * BPW
        copy = pltpu.async_copy(embed_hbm, embed_v, sem)
        pltpu.sync_copy(idx_hbm.at[pl.ds(b * HW + hw0, BPW)], idx_v)
        copy.wait()

        def per_group(g, _):
            idx16 = idx_v[pl.ds(g * 16, 16)]

            def per_dim(d, _):
                dvec = jnp.full((16,), 0, jnp.int32) + d
                vals_v[d, pl.ds(g * 16, 16)] = plsc.load_gather(
                    embed_v, [dvec, idx16])
                return 0

            lax.fori_loop(0, DIM, per_dim, 0, unroll=8)
            return 0

        lax.fori_loop(0, BPW // 16, per_group, 0)
        pltpu.sync_copy(vals_v, out_hbm.at[b, :, pl.ds(hw0, BPW)])

    return k(idx_flat, embed)


def kernel(x, embed):
    x3 = x.reshape(BATCH, DIM, HW)
    ind3, diff3, perp2 = _vq_tc(x3, embed)
    quantize = _sc_gather(ind3.reshape(N_TOKENS), embed).reshape(
        BATCH, DIM, 32, 32)
    return quantize, diff3.reshape(BATCH, 1), perp2.reshape(())


# speedup vs baseline: 1.4259x; 1.0015x over previous
import functools

import jax
import jax.numpy as jnp
from jax import lax
from jax.experimental import pallas as pl
from jax.experimental.pallas import tpu as pltpu
from jax.experimental.pallas import tpu_sc as plsc

DIM = 64
N_CODES = 1024
BATCH = 16
HW = 1024
N_TOKENS = BATCH * HW
INV_TOKENS = 1.0 / N_TOKENS
INV_HWD = 1.0 / (HW * DIM)

NW = 32
BPW = N_TOKENS // NW


def _vq_tc_body(x_ref, embed_ref, ind_ref, diff_ref, perp_ref, counts_ref):
    b = pl.program_id(0)
    x_b = x_ref[0]
    embed = embed_ref[...]
    z2 = jnp.sum(x_b * x_b, axis=0)
    e2 = jnp.sum(embed * embed, axis=0)
    prod = lax.dot_general(embed, x_b, (((0,), (0,)), ((), ())),
                           preferred_element_type=jnp.float32)
    neg = (2.0 * prod - z2[None, :]) - e2[:, None]
    m = jnp.max(neg, axis=0)
    ind = jnp.argmax(neg, axis=0).astype(jnp.int32)
    ind_ref[...] = ind
    diff_ref[0] = jnp.sum(-m.reshape(1, HW), axis=1, keepdims=True) * INV_HWD
    onehot = (neg == m[None, :]).astype(jnp.float32)
    counts = lax.dot_general(onehot, jnp.ones((HW, 1), jnp.float32),
                             (((1,), (0,)), ((), ())),
                             preferred_element_type=jnp.float32)

    @pl.when(b == 0)
    def _init():
        counts_ref[...] = counts

    @pl.when(b > 0)
    def _acc():
        counts_ref[...] = counts_ref[...] + counts

    @pl.when(b == BATCH - 1)
    def _fin():
        avg = counts_ref[...] * INV_TOKENS
        ent = jnp.sum(avg * jnp.log(avg + 1e-10), axis=0, keepdims=True)
        perp_ref[...] = jnp.exp(-ent)


def _vq_tc(x3, embed):
    return pl.pallas_call(
        _vq_tc_body,
        grid=(BATCH,),
        in_specs=[
            pl.BlockSpec((1, DIM, HW), lambda b: (b, 0, 0)),
            pl.BlockSpec((DIM, N_CODES), lambda b: (0, 0)),
        ],
        out_specs=[
            pl.BlockSpec((HW,), lambda b: (b,)),
            pl.BlockSpec((1, 1, 1), lambda b: (b, 0, 0)),
            pl.BlockSpec((1, 1), lambda b: (0, 0)),
        ],
        out_shape=[
            jax.ShapeDtypeStruct((N_TOKENS,), jnp.int32),
            jax.ShapeDtypeStruct((BATCH, 1, 1), jnp.float32),
            jax.ShapeDtypeStruct((1, 1), jnp.float32),
        ],
        scratch_shapes=[pltpu.VMEM((N_CODES, 1), jnp.float32)],
        compiler_params=pltpu.CompilerParams(allow_input_fusion=[0]),
    )(x3, embed)


def _sc_gather(idx_flat, embed):
    mesh = plsc.VectorSubcoreMesh(core_axis_name="c", subcore_axis_name="s")

    @functools.partial(
        pl.kernel, mesh=mesh,
        out_type=jax.ShapeDtypeStruct((BATCH, DIM, HW), jnp.float32),
        scratch_types=[
            pltpu.VMEM((DIM, N_CODES), jnp.float32),
            pltpu.VMEM((BPW,), jnp.int32),
            pltpu.VMEM((DIM, BPW), jnp.float32),
            pltpu.SemaphoreType.DMA,
        ],
        compiler_params=pltpu.CompilerParams(
            use_tc_tiling_on_sc=False, needs_layout_passes=False),
    )
    def k(idx_hbm, embed_hbm, out_hbm, embed_v, idx_v, vals_v, sem):
        wid = lax.axis_index("s") * 2 + lax.axis_index("c")
        b = wid // 2
        hw0 = (wid % 2) * BPW
        copy = pltpu.async_copy(embed_hbm, embed_v, sem)
        pltpu.sync_copy(idx_hbm.at[pl.ds(b * HW + hw0, BPW)], idx_v)
        copy.wait()

        def per_group(g, _):
            idx16 = idx_v[pl.ds(g * 16, 16)]

            def per_dim(d, _):
                dvec = jnp.full((16,), 0, jnp.int32) + d
                vals_v[d, pl.ds(g * 16, 16)] = plsc.load_gather(
                    embed_v, [dvec, idx16])
                return 0

            lax.fori_loop(0, DIM, per_dim, 0, unroll=8)
            return 0

        lax.fori_loop(0, BPW // 16, per_group, 0)
        pltpu.sync_copy(vals_v, out_hbm.at[b, :, pl.ds(hw0, BPW)])

    return k(idx_flat, embed)


def kernel(x, embed):
    x3 = x.reshape(BATCH, DIM, HW)
    ind1, diff3, perp2 = _vq_tc(x3, embed)
    quantize = _sc_gather(ind1, embed).reshape(BATCH, DIM, 32, 32)
    return quantize, diff3.reshape(BATCH, 1), perp2.reshape(())
